# baseline (speedup 1.0000x reference)
import functools

import jax
import jax.numpy as jnp
from jax import lax
from jax.experimental import pallas as pl
from jax.experimental.pallas import tpu as pltpu
from jax.experimental.pallas import tpu_sc as plsc

N_NODES = 10000
N_EDGES = 320000
HIDDEN = 128
NUM_RADIAL = 16


_EDGE_BLK = 3200
_PACK = 128 // NUM_RADIAL
_PBLK = _EDGE_BLK // _PACK
_PH = _PACK * HIDDEN


def _radial_body(rb_ref, bd1_ref, b1p_ref, w2_ref, b2p_ref, out_ref):
    x = jnp.dot(rb_ref[...], bd1_ref[...], preferred_element_type=jnp.float32)
    x = x + b1p_ref[...]
    x = x * jax.nn.sigmoid(x)
    xb = x.astype(jnp.bfloat16)
    for g in range(_PACK):
        sl = slice(g * HIDDEN, (g + 1) * HIDDEN)
        y = jnp.dot(xb[:, sl], w2_ref[...], preferred_element_type=jnp.float32)
        out_ref[:, sl] = y + b2p_ref[:, sl]


def _radial_mlp(rb_packed, BD1, b1p, W2, b2p):
    grid = (N_EDGES // _EDGE_BLK,)
    out = pl.pallas_call(
        _radial_body,
        grid=grid,
        in_specs=[
            pl.BlockSpec((_PBLK, 128), lambda i: (i, 0)),
            pl.BlockSpec((128, _PH), lambda i: (0, 0)),
            pl.BlockSpec((1, _PH), lambda i: (0, 0)),
            pl.BlockSpec((HIDDEN, HIDDEN), lambda i: (0, 0)),
            pl.BlockSpec((1, _PH), lambda i: (0, 0)),
        ],
        out_specs=pl.BlockSpec((_PBLK, _PH), lambda i: (i, 0)),
        out_shape=jax.ShapeDtypeStruct((N_EDGES // _PACK, _PH), jnp.float32),
    )(rb_packed, BD1, b1p, W2, b2p)
    return out.reshape(N_EDGES, HIDDEN)


_NC = 2
_NS = 16
_NW = _NC * _NS
_EPW = N_EDGES // _NW
_CHUNK = 80
_STEPS = _EPW // _CHUNK
_RPT = 624


def _sc_body(h_hbm, w_hbm, idx_hbm, out_hbm,
             idx0_v, idx1_v, rows0_v, rows1_v, w0_v, w1_v, acc_sh,
             sem_g0, sem_g1, sem_w0, sem_w1):
    c = lax.axis_index("c")
    s = lax.axis_index("s")
    wid = c * _NS + s
    idx_bufs = (idx0_v, idx1_v)
    rows_bufs = (rows0_v, rows1_v)
    w_bufs = (w0_v, w1_v)
    sems_g = (sem_g0, sem_g1)
    sems_w = (sem_w0, sem_w1)

    def _fetch_issue(t, b):
        pltpu.sync_copy(idx_hbm.at[:, wid, t], idx_bufs[b])
        pltpu.make_async_copy(h_hbm.at[idx_bufs[b].at[0, 0]], rows_bufs[b],
                              sems_g[b]).start()
        pltpu.make_async_copy(w_hbm.at[wid, pl.ds(t * _CHUNK, _CHUNK)],
                              w_bufs[b], sems_w[b]).start()

    def _zrow(i, carry):
        for j in range(HIDDEN // 16):
            rows0_v[i, pl.ds(j * 16, 16)] = jnp.zeros((16,), jnp.float32)
        return carry
    lax.fori_loop(0, _CHUNK, _zrow, 0)
    zbase = s * _RPT
    for k in range(_RPT // _CHUNK):
        pltpu.sync_copy(rows0_v.at[pl.ds(0, _CHUNK)],
                        acc_sh.at[pl.ds(zbase + k * _CHUNK, _CHUNK)])
    _rem = _RPT % _CHUNK
    if _rem:
        pltpu.sync_copy(rows0_v.at[pl.ds(0, _rem)],
                        acc_sh.at[pl.ds(zbase + _RPT - _rem, _rem)])
    _tail = N_NODES - _NS * _RPT

    @pl.when(s == _NS - 1)
    def _ztail():
        pltpu.sync_copy(rows0_v.at[pl.ds(0, _tail)],
                        acc_sh.at[pl.ds(_NS * _RPT, _tail)])

    _fetch_issue(0, 0)
    _fetch_issue(1, 1)
    plsc.subcore_barrier()

    def _do_step(t, b):
        rows_b, w_b = rows_bufs[b], w_bufs[b]
        pltpu.make_async_copy(h_hbm.at[idx_bufs[b].at[0, 0]], rows_b,
                              sems_g[b]).wait()
        pltpu.make_async_copy(w_hbm.at[wid, pl.ds(t * _CHUNK, _CHUNK)],
                              w_b, sems_w[b]).wait()

        def _mul(i, carry2):
            for j in range(HIDDEN // 16):
                sl = pl.ds(j * 16, 16)
                rows_b[i, sl] = rows_b[i, sl] * w_b[i, sl]
            return carry2
        lax.fori_loop(0, _CHUNK, _mul, 0)

        pltpu.sync_copy(rows_b, acc_sh.at[idx_bufs[b].at[1, 0]], add=True)

    def _pair(g, carry):
        for b in range(2):
            t = 2 * g + b

            _do_step(t, b)

            @pl.when(t + 2 < _STEPS)
            def _prefetch():
                _fetch_issue(t + 2, b)
        return carry
    lax.fori_loop(0, _STEPS // 2, _pair, 0)
    if _STEPS % 2:
        _do_step(_STEPS - 1, 0)
    plsc.subcore_barrier()

    for k in range(_RPT // _CHUNK):
        off = zbase + k * _CHUNK
        pltpu.sync_copy(acc_sh.at[pl.ds(off, _CHUNK)],
                        out_hbm.at[c, pl.ds(off, _CHUNK)])
    if _rem:
        off = zbase + _RPT - _rem
        pltpu.sync_copy(acc_sh.at[pl.ds(off, _rem)],
                        out_hbm.at[c, pl.ds(off, _rem)])

    @pl.when(s == _NS - 1)
    def _wtail():
        pltpu.sync_copy(acc_sh.at[pl.ds(_NS * _RPT, _tail)],
                        out_hbm.at[c, pl.ds(_NS * _RPT, _tail)])


def _sc_gather_mul_scatter(h, w_edges, ei):
    mesh = plsc.VectorSubcoreMesh(core_axis_name="c", subcore_axis_name="s")
    f = functools.partial(
        pl.kernel,
        mesh=mesh,
        out_type=jax.ShapeDtypeStruct((_NC, N_NODES, HIDDEN), jnp.float32),
        scratch_types=[
            pltpu.VMEM((2, 1, _CHUNK), jnp.int32),
            pltpu.VMEM((2, 1, _CHUNK), jnp.int32),
            pltpu.VMEM((_CHUNK, HIDDEN), jnp.float32),
            pltpu.VMEM((_CHUNK, HIDDEN), jnp.float32),
            pltpu.VMEM((_CHUNK, HIDDEN), jnp.float32),
            pltpu.VMEM((_CHUNK, HIDDEN), jnp.float32),
            pltpu.VMEM_SHARED((N_NODES, HIDDEN), jnp.float32),
            pltpu.SemaphoreType.DMA,
            pltpu.SemaphoreType.DMA,
            pltpu.SemaphoreType.DMA,
            pltpu.SemaphoreType.DMA,
        ],
    )(_sc_body)
    idx2 = ei.reshape(2, _NW, _STEPS, 1, _CHUNK)
    return f(h, w_edges.reshape(_NW, _EPW, HIDDEN), idx2)


_NODE_BLK = 1000


def _mlp_body(p_ref, wd0_ref, bd0_ref, wd1_ref, bd1_ref, wd2_ref, bd2_ref,
              wo_ref, bo_ref, out_ref):
    x = p_ref[0] + p_ref[1]
    for w_ref, b_ref in ((wd0_ref, bd0_ref), (wd1_ref, bd1_ref), (wd2_ref, bd2_ref)):
        x = jnp.dot(x, w_ref[...], preferred_element_type=jnp.float32) + b_ref[...]
        x = x * jax.nn.sigmoid(x)
    out_ref[...] = (jnp.dot(x, wo_ref[...], preferred_element_type=jnp.float32)
                    + bo_ref[...])


def _node_mlp(partials, Wd0, bd0, Wd1, bd1, Wd2, bd2, Wo, bo):
    grid = (N_NODES // _NODE_BLK,)
    return pl.pallas_call(
        _mlp_body,
        grid=grid,
        in_specs=[
            pl.BlockSpec((_NC, _NODE_BLK, HIDDEN), lambda i: (0, i, 0)),
            pl.BlockSpec((HIDDEN, HIDDEN), lambda i: (0, 0)),
            pl.BlockSpec((1, HIDDEN), lambda i: (0, 0)),
            pl.BlockSpec((HIDDEN, HIDDEN), lambda i: (0, 0)),
            pl.BlockSpec((1, HIDDEN), lambda i: (0, 0)),
            pl.BlockSpec((HIDDEN, HIDDEN), lambda i: (0, 0)),
            pl.BlockSpec((1, HIDDEN), lambda i: (0, 0)),
            pl.BlockSpec((HIDDEN, 1), lambda i: (0, 0)),
            pl.BlockSpec((1, 1), lambda i: (0, 0)),
        ],
        out_specs=pl.BlockSpec((_NODE_BLK, 1), lambda i: (i, 0)),
        out_shape=jax.ShapeDtypeStruct((N_NODES, 1), jnp.float32),
    )(partials, Wd0, bd0, Wd1, bd1, Wd2, bd2, Wo, bo)


def kernel(h, radial_basis, edge_index, W1, b1, W2, b2,
           Wd0, bd0, Wd1, bd1, Wd2, bd2, Wo, bo):
    ei = edge_index.astype(jnp.int32)
    BD1 = (jnp.einsum("gh,kj->gkhj", jnp.eye(_PACK, dtype=jnp.float32),
                      W1).reshape(128, _PH))
    b1p = jnp.tile(b1, _PACK).reshape(1, _PH)
    b2p = jnp.tile(b2, _PACK).reshape(1, _PH)
    w_edges = _radial_mlp(radial_basis.reshape(N_EDGES // _PACK, 128),
                          BD1, b1p, W2.astype(jnp.bfloat16), b2p)
    partials = _sc_gather_mul_scatter(h, w_edges, ei)
    return _node_mlp(partials, Wd0, bd0.reshape(1, HIDDEN),
                     Wd1, bd1.reshape(1, HIDDEN), Wd2, bd2.reshape(1, HIDDEN),
                     Wo, bo.reshape(1, 1))


# speedup vs baseline: 1.6438x; 1.6438x over previous
import functools

import jax
import jax.numpy as jnp
from jax import lax
from jax.experimental import pallas as pl
from jax.experimental.pallas import tpu as pltpu
from jax.experimental.pallas import tpu_sc as plsc

N_NODES = 10000
N_EDGES = 320000
HIDDEN = 128
NUM_RADIAL = 16


_EDGE_BLK = 2560


def _radial_body(rbt_ref, w1_ref, b1_ref, w2_ref, b2_ref, out_ref):
    x = lax.dot_general(rbt_ref[...], w1_ref[...],
                        ((( 0,), (0,)), ((), ())),
                        preferred_element_type=jnp.float32)
    x = x + b1_ref[...]
    x = x * jax.nn.sigmoid(x)
    y = jnp.dot(x.astype(jnp.bfloat16), w2_ref[...],
                preferred_element_type=jnp.float32)
    out_ref[...] = y + b2_ref[...]


def _radial_mlp(rbt, W1, b1, W2, b2):
    grid = (N_EDGES // _EDGE_BLK,)
    return pl.pallas_call(
        _radial_body,
        grid=grid,
        in_specs=[
            pl.BlockSpec((NUM_RADIAL, _EDGE_BLK), lambda i: (0, i)),
            pl.BlockSpec((NUM_RADIAL, HIDDEN), lambda i: (0, 0)),
            pl.BlockSpec((1, HIDDEN), lambda i: (0, 0)),
            pl.BlockSpec((HIDDEN, HIDDEN), lambda i: (0, 0)),
            pl.BlockSpec((1, HIDDEN), lambda i: (0, 0)),
        ],
        out_specs=pl.BlockSpec((_EDGE_BLK, HIDDEN), lambda i: (i, 0)),
        out_shape=jax.ShapeDtypeStruct((N_EDGES, HIDDEN), jnp.float32),
    )(rbt, W1, b1, W2, b2)


_NC = 2
_NS = 16
_NW = _NC * _NS
_EPW = N_EDGES // _NW
_CHUNK = 80
_STEPS = _EPW // _CHUNK
_RPT = 624


def _sc_body(h_hbm, w_hbm, idx_hbm, out_hbm,
             idx0_v, idx1_v, rows0_v, rows1_v, w0_v, w1_v, acc_sh,
             sem_g0, sem_g1, sem_w0, sem_w1):
    c = lax.axis_index("c")
    s = lax.axis_index("s")
    wid = c * _NS + s
    idx_bufs = (idx0_v, idx1_v)
    rows_bufs = (rows0_v, rows1_v)
    w_bufs = (w0_v, w1_v)
    sems_g = (sem_g0, sem_g1)
    sems_w = (sem_w0, sem_w1)

    def _fetch_issue(t, b):
        pltpu.sync_copy(idx_hbm.at[:, wid, t], idx_bufs[b])
        pltpu.make_async_copy(h_hbm.at[idx_bufs[b].at[0, 0]], rows_bufs[b],
                              sems_g[b]).start()
        pltpu.make_async_copy(w_hbm.at[wid, pl.ds(t * _CHUNK, _CHUNK)],
                              w_bufs[b], sems_w[b]).start()

    def _zrow(i, carry):
        for j in range(HIDDEN // 16):
            rows0_v[i, pl.ds(j * 16, 16)] = jnp.zeros((16,), jnp.float32)
        return carry
    lax.fori_loop(0, _CHUNK, _zrow, 0)
    zbase = s * _RPT
    for k in range(_RPT // _CHUNK):
        pltpu.sync_copy(rows0_v.at[pl.ds(0, _CHUNK)],
                        acc_sh.at[pl.ds(zbase + k * _CHUNK, _CHUNK)])
    _rem = _RPT % _CHUNK
    if _rem:
        pltpu.sync_copy(rows0_v.at[pl.ds(0, _rem)],
                        acc_sh.at[pl.ds(zbase + _RPT - _rem, _rem)])
    _tail = N_NODES - _NS * _RPT

    @pl.when(s == _NS - 1)
    def _ztail():
        pltpu.sync_copy(rows0_v.at[pl.ds(0, _tail)],
                        acc_sh.at[pl.ds(_NS * _RPT, _tail)])

    _fetch_issue(0, 0)
    _fetch_issue(1, 1)
    plsc.subcore_barrier()

    def _do_step(t, b):
        rows_b, w_b = rows_bufs[b], w_bufs[b]
        pltpu.make_async_copy(h_hbm.at[idx_bufs[b].at[0, 0]], rows_b,
                              sems_g[b]).wait()
        pltpu.make_async_copy(w_hbm.at[wid, pl.ds(t * _CHUNK, _CHUNK)],
                              w_b, sems_w[b]).wait()

        def _mul(i, carry2):
            for j in range(HIDDEN // 16):
                sl = pl.ds(j * 16, 16)
                rows_b[i, sl] = rows_b[i, sl] * w_b[i, sl]
            return carry2
        lax.fori_loop(0, _CHUNK, _mul, 0)

        pltpu.sync_copy(rows_b, acc_sh.at[idx_bufs[b].at[1, 0]], add=True)

    def _pair(g, carry):
        for b in range(2):
            t = 2 * g + b

            _do_step(t, b)

            @pl.when(t + 2 < _STEPS)
            def _prefetch():
                _fetch_issue(t + 2, b)
        return carry
    lax.fori_loop(0, _STEPS // 2, _pair, 0)
    if _STEPS % 2:
        _do_step(_STEPS - 1, 0)
    plsc.subcore_barrier()

    for k in range(_RPT // _CHUNK):
        off = zbase + k * _CHUNK
        pltpu.sync_copy(acc_sh.at[pl.ds(off, _CHUNK)],
                        out_hbm.at[c, pl.ds(off, _CHUNK)])
    if _rem:
        off = zbase + _RPT - _rem
        pltpu.sync_copy(acc_sh.at[pl.ds(off, _rem)],
                        out_hbm.at[c, pl.ds(off, _rem)])

    @pl.when(s == _NS - 1)
    def _wtail():
        pltpu.sync_copy(acc_sh.at[pl.ds(_NS * _RPT, _tail)],
                        out_hbm.at[c, pl.ds(_NS * _RPT, _tail)])


def _sc_gather_mul_scatter(h, w_edges, ei):
    mesh = plsc.VectorSubcoreMesh(core_axis_name="c", subcore_axis_name="s")
    f = functools.partial(
        pl.kernel,
        mesh=mesh,
        out_type=jax.ShapeDtypeStruct((_NC, N_NODES, HIDDEN), jnp.float32),
        scratch_types=[
            pltpu.VMEM((2, 1, _CHUNK), jnp.int32),
            pltpu.VMEM((2, 1, _CHUNK), jnp.int32),
            pltpu.VMEM((_CHUNK, HIDDEN), jnp.float32),
            pltpu.VMEM((_CHUNK, HIDDEN), jnp.float32),
            pltpu.VMEM((_CHUNK, HIDDEN), jnp.float32),
            pltpu.VMEM((_CHUNK, HIDDEN), jnp.float32),
            pltpu.VMEM_SHARED((N_NODES, HIDDEN), jnp.float32),
            pltpu.SemaphoreType.DMA,
            pltpu.SemaphoreType.DMA,
            pltpu.SemaphoreType.DMA,
            pltpu.SemaphoreType.DMA,
        ],
    )(_sc_body)
    idx2 = ei.reshape(2, _NW, _STEPS, 1, _CHUNK)
    return f(h, w_edges.reshape(_NW, _EPW, HIDDEN), idx2)


_NODE_BLK = 1000


def _mlp_body(p_ref, wd0_ref, bd0_ref, wd1_ref, bd1_ref, wd2_ref, bd2_ref,
              wo_ref, bo_ref, out_ref):
    x = p_ref[0] + p_ref[1]
    for w_ref, b_ref in ((wd0_ref, bd0_ref), (wd1_ref, bd1_ref), (wd2_ref, bd2_ref)):
        x = jnp.dot(x, w_ref[...], preferred_element_type=jnp.float32) + b_ref[...]
        x = x * jax.nn.sigmoid(x)
    out_ref[...] = (jnp.dot(x, wo_ref[...], preferred_element_type=jnp.float32)
                    + bo_ref[...])


def _node_mlp(partials, Wd0, bd0, Wd1, bd1, Wd2, bd2, Wo, bo):
    grid = (N_NODES // _NODE_BLK,)
    return pl.pallas_call(
        _mlp_body,
        grid=grid,
        in_specs=[
            pl.BlockSpec((_NC, _NODE_BLK, HIDDEN), lambda i: (0, i, 0)),
            pl.BlockSpec((HIDDEN, HIDDEN), lambda i: (0, 0)),
            pl.BlockSpec((1, HIDDEN), lambda i: (0, 0)),
            pl.BlockSpec((HIDDEN, HIDDEN), lambda i: (0, 0)),
            pl.BlockSpec((1, HIDDEN), lambda i: (0, 0)),
            pl.BlockSpec((HIDDEN, HIDDEN), lambda i: (0, 0)),
            pl.BlockSpec((1, HIDDEN), lambda i: (0, 0)),
            pl.BlockSpec((HIDDEN, 1), lambda i: (0, 0)),
            pl.BlockSpec((1, 1), lambda i: (0, 0)),
        ],
        out_specs=pl.BlockSpec((_NODE_BLK, 1), lambda i: (i, 0)),
        out_shape=jax.ShapeDtypeStruct((N_NODES, 1), jnp.float32),
    )(partials, Wd0, bd0, Wd1, bd1, Wd2, bd2, Wo, bo)


def kernel(h, radial_basis, edge_index, W1, b1, W2, b2,
           Wd0, bd0, Wd1, bd1, Wd2, bd2, Wo, bo):
    ei = edge_index.astype(jnp.int32)
    w_edges = _radial_mlp(radial_basis.T, W1, b1.reshape(1, HIDDEN),
                          W2.astype(jnp.bfloat16), b2.reshape(1, HIDDEN))
    partials = _sc_gather_mul_scatter(h, w_edges, ei)
    return _node_mlp(partials, Wd0, bd0.reshape(1, HIDDEN),
                     Wd1, bd1.reshape(1, HIDDEN), Wd2, bd2.reshape(1, HIDDEN),
                     Wo, bo.reshape(1, 1))
